# baseline (speedup 1.0000x reference)
import functools

import jax
import jax.numpy as jnp
import numpy as np
from jax import lax
from jax.experimental import pallas as pl
from jax.experimental.pallas import tpu as pltpu
from jax.experimental.pallas import tpu_sc as plsc

B = 2
PP = 4096
D = 768
K = 8
R = 512
C = 1024
NEG_MASK = np.float32(-1e9)
NEG_INIT = np.float32(-1e30)
BIGI = np.int32(2 ** 30)


def _proj_body(x_ref, wq_ref, bq_ref, wk_ref, bk_ref, wv_ref, bv_ref,
               q_ref, k_ref, v_ref):
    x = x_ref[0]

    def proj(w_ref, b_ref):
        y = lax.dot_general(x, w_ref[...], (((1,), (1,)), ((), ())),
                            preferred_element_type=jnp.float32)
        return y + b_ref[...]

    def norm(a):
        n = jnp.sqrt(jnp.sum(a * a, axis=-1, keepdims=True))
        return a / jnp.maximum(n, 1e-12)

    q_ref[0] = norm(proj(wq_ref, bq_ref))
    k_ref[0] = norm(proj(wk_ref, bk_ref))
    v_ref[0] = proj(wv_ref, bv_ref)


def _project(x, Wq, bq, Wk, bk, Wv, bv):
    grid = (B, PP // R)
    blk_w = pl.BlockSpec((D, D), lambda b, i: (0, 0))
    blk_b = pl.BlockSpec((1, D), lambda b, i: (0, 0))
    blk_row = pl.BlockSpec((1, R, D), lambda b, i: (b, i, 0))
    out_sd = jax.ShapeDtypeStruct((B, PP, D), jnp.float32)
    return pl.pallas_call(
        _proj_body,
        grid=grid,
        in_specs=[blk_row, blk_w, blk_b, blk_w, blk_b, blk_w, blk_b],
        out_specs=[blk_row, blk_row, blk_row],
        out_shape=[out_sd, out_sd, out_sd],
    )(x, Wq, bq.reshape(1, D), Wk, bk.reshape(1, D), Wv, bv.reshape(1, D))


def _topk_body(q_ref, k_ref, routes_ref, weights_ref, vals_ref, idx_ref,
               jloc_ref):
    rt = pl.program_id(0)
    ct = pl.program_id(1)
    r0 = rt * R
    c0 = ct * C

    @pl.when((rt == 0) & (ct == 0))
    def _():
        jloc_ref[...] = lax.broadcasted_iota(jnp.int32, (R, C), 1)

    @pl.when(ct == 0)
    def _():
        vals_ref[...] = jnp.full((R, K), NEG_INIT, jnp.float32)
        idx_ref[...] = jnp.full((R, K), BIGI, jnp.int32)

    q = q_ref[0]
    k = k_ref[0]
    s = lax.dot_general(q, k, (((1,), (1,)), ((), ())),
                        preferred_element_type=jnp.float32)
    jloc = jloc_ref[...]
    ii = lax.broadcasted_iota(jnp.int32, (R, 1), 0)
    s = jnp.where((jloc == ii + (r0 - c0)) | (jloc == -c0), NEG_MASK, s)

    tv, ti = [], []
    for _ in range(K):
        m = jnp.max(s, axis=1, keepdims=True)
        t = jnp.where(s == m, jloc, BIGI)
        sel = jnp.min(t, axis=1, keepdims=True)
        tv.append(m)
        ti.append(sel)
        s = jnp.where(t == sel, NEG_INIT, s)
    tile_v = jnp.concatenate(tv, axis=1)
    tile_i = jnp.concatenate(ti, axis=1) + (c0 - 1)

    cv = jnp.concatenate([vals_ref[...], tile_v], axis=1)
    ci = jnp.concatenate([idx_ref[...], tile_i], axis=1)
    nv, ni = [], []
    for _ in range(K):
        m = jnp.max(cv, axis=1, keepdims=True)
        t = jnp.where(cv == m, ci, BIGI)
        sel = jnp.min(t, axis=1, keepdims=True)
        nv.append(m)
        ni.append(sel)
        cv = jnp.where(t == sel, NEG_INIT, cv)
    vals_ref[...] = jnp.concatenate(nv, axis=1)
    idx_ref[...] = jnp.concatenate(ni, axis=1)

    @pl.when(ct == pl.num_programs(1) - 1)
    def _():
        tv8 = vals_ref[...] / jnp.float32(0.1)
        mx = jnp.max(tv8, axis=1, keepdims=True)
        e = jnp.exp(tv8 - mx)
        weights_ref[0] = e / jnp.sum(e, axis=1, keepdims=True)
        routes_ref[0] = idx_ref[...]


def _topk_route(q, k, b):
    grid = (PP // R, PP // C)
    return pl.pallas_call(
        _topk_body,
        grid=grid,
        in_specs=[
            pl.BlockSpec((1, R, D), lambda rt, ct: (b, rt, 0)),
            pl.BlockSpec((1, C, D), lambda rt, ct: (b, ct, 0)),
        ],
        out_specs=[
            pl.BlockSpec((1, R, K), lambda rt, ct: (0, rt, 0)),
            pl.BlockSpec((1, R, K), lambda rt, ct: (0, rt, 0)),
        ],
        out_shape=[
            jax.ShapeDtypeStruct((1, PP, K), jnp.int32),
            jax.ShapeDtypeStruct((1, PP, K), jnp.float32),
        ],
        scratch_shapes=[
            pltpu.VMEM((R, K), jnp.float32),
            pltpu.VMEM((R, K), jnp.int32),
            pltpu.VMEM((R, C), jnp.int32),
        ],
    )(q, k)


SC_ROWS = 4096
SC_NW = 32
SC_RPW = SC_ROWS // SC_NW
SC_CH = 8
SC_NCH = SC_RPW // SC_CH


def _lane_bcast(vec, lane):
    idx = jnp.full((16, 1), lane, jnp.int32)
    dn = lax.GatherDimensionNumbers(
        offset_dims=(), collapsed_slice_dims=(0,), start_index_map=(0,))
    return lax.gather(vec, idx, dn, (1,),
                      mode=lax.GatherScatterMode.PROMISE_IN_BOUNDS)


def _sc_body(v_hbm, gidx_hbm, w_hbm, out_hbm,
             idx_all, w_all, rows_a, rows_b, acc_v, sem_a, sem_b):
    nc = 2
    wid = lax.axis_index("s") * nc + lax.axis_index("c")
    base_row = wid * SC_RPW
    ibase = base_row * K
    pltpu.sync_copy(gidx_hbm.at[pl.ds(ibase, SC_RPW * K)], idx_all)
    pltpu.sync_copy(w_hbm.at[pl.ds(ibase, SC_RPW * K)], w_all)

    def start_gather(ci, rows_ref, sem):
        pltpu.async_copy(
            v_hbm.at[idx_all.at[pl.ds(ci * (SC_CH * K), SC_CH * K)]],
            rows_ref, sem)

    def drain(rows_ref, sem):
        pltpu.make_async_copy(v_hbm.at[pl.ds(0, SC_CH * K)], rows_ref,
                              sem).wait()

    def compute(ci, rows_ref):
        def row_body(r, _):
            wvec = w_all[pl.ds(ci * (SC_CH * K) + (r // 2) * 16, 16)]
            half = (r % 2) * K
            wk = [_lane_bcast(wvec, half + kk) for kk in range(K)]

            def dc_body(dc, _):
                a = rows_ref[r * K, pl.ds(dc * 16, 16)] * wk[0]
                for kk in range(1, K):
                    a = a + rows_ref[r * K + kk, pl.ds(dc * 16, 16)] * wk[kk]
                acc_v[r, pl.ds(dc * 16, 16)] = a
                return 0

            lax.fori_loop(0, D // 16, dc_body, 0, unroll=4)
            return 0

        lax.fori_loop(0, SC_CH, row_body, 0)
        rbase = base_row + ci * SC_CH
        pltpu.sync_copy(acc_v, out_hbm.at[pl.ds(rbase, SC_CH)])

    start_gather(0, rows_a, sem_a)

    def pair_body(i, _):
        ca = 2 * i
        cb = 2 * i + 1
        start_gather(cb, rows_b, sem_b)
        drain(rows_a, sem_a)
        compute(ca, rows_a)
        nxt = lax.rem(ca + 2, SC_NCH)
        start_gather(nxt, rows_a, sem_a)
        drain(rows_b, sem_b)
        compute(cb, rows_b)
        return 0

    lax.fori_loop(0, SC_NCH // 2, pair_body, 0)
    drain(rows_a, sem_a)


def _sc_combine(v_flat, gidx, w_flat):
    mesh = plsc.VectorSubcoreMesh(core_axis_name="c", subcore_axis_name="s")
    fn = functools.partial(
        pl.kernel,
        mesh=mesh,
        out_type=jax.ShapeDtypeStruct((SC_ROWS, D), jnp.float32),
        scratch_types=[
            pltpu.VMEM((SC_RPW * K,), jnp.int32),
            pltpu.VMEM((SC_RPW * K,), jnp.float32),
            pltpu.VMEM((SC_CH * K, D), jnp.float32),
            pltpu.VMEM((SC_CH * K, D), jnp.float32),
            pltpu.VMEM((SC_CH, D), jnp.float32),
            pltpu.SemaphoreType.DMA,
            pltpu.SemaphoreType.DMA,
        ],
    )(_sc_body)
    return fn(v_flat, gidx, w_flat)


def kernel(x, Wq, bq, Wk, bk, Wv, bv):
    q, k, v = _project(x, Wq, bq, Wk, bk, Wv, bv)
    v_flat = v.reshape(B * PP, D)

    routes_b, weights_b, feats_b = [], [], []
    for b in range(B):
        r_full, w_full = _topk_route(q, k, b)
        routes1 = r_full[0, 1:, :]
        weights1 = w_full[0, 1:, :]
        gidx = jnp.pad((routes1 + (b * PP + 1)).reshape(-1), (0, K))
        w_flat = jnp.pad(weights1.reshape(-1), (0, K))
        feat = _sc_combine(v_flat, gidx, w_flat)
        routes_b.append(routes1)
        weights_b.append(weights1)
        feats_b.append(feat[: PP - 1])

    routes = jnp.stack(routes_b)
    weights = jnp.stack(weights_b)
    features = jnp.stack(feats_b)
    return (routes, weights, features)


# speedup vs baseline: 1.3929x; 1.3929x over previous
import functools

import jax
import jax.numpy as jnp
import numpy as np
from jax import lax
from jax.experimental import pallas as pl
from jax.experimental.pallas import tpu as pltpu
from jax.experimental.pallas import tpu_sc as plsc

B = 2
PP = 4096
D = 768
K = 8
R = 512
C = 4096
NEG_MASK = np.float32(-1e9)
NEG_INIT = np.float32(-1e30)
BIGI = np.int32(2 ** 30)


def _proj_body(x_ref, wq_ref, bq_ref, wk_ref, bk_ref, wv_ref, bv_ref,
               q_ref, k_ref, v_ref):
    x = x_ref[0]

    def proj(w_ref, b_ref):
        y = lax.dot_general(x, w_ref[...], (((1,), (1,)), ((), ())),
                            preferred_element_type=jnp.float32)
        return y + b_ref[...]

    def norm(a):
        n = jnp.sqrt(jnp.sum(a * a, axis=-1, keepdims=True))
        return a / jnp.maximum(n, 1e-12)

    q_ref[0] = norm(proj(wq_ref, bq_ref))
    k_ref[0] = norm(proj(wk_ref, bk_ref))
    v_ref[0] = proj(wv_ref, bv_ref)


def _project(x, Wq, bq, Wk, bk, Wv, bv):
    grid = (B, PP // R)
    blk_w = pl.BlockSpec((D, D), lambda b, i: (0, 0))
    blk_b = pl.BlockSpec((1, D), lambda b, i: (0, 0))
    blk_row = pl.BlockSpec((1, R, D), lambda b, i: (b, i, 0))
    out_sd = jax.ShapeDtypeStruct((B, PP, D), jnp.float32)
    return pl.pallas_call(
        _proj_body,
        grid=grid,
        in_specs=[blk_row, blk_w, blk_b, blk_w, blk_b, blk_w, blk_b],
        out_specs=[blk_row, blk_row, blk_row],
        out_shape=[out_sd, out_sd, out_sd],
    )(x, Wq, bq.reshape(1, D), Wk, bk.reshape(1, D), Wv, bv.reshape(1, D))


def _topk_body(q_ref, k_ref, routes_ref, weights_ref):
    rt = pl.program_id(0)
    r0 = rt * R

    q = q_ref[0]
    k = k_ref[0]
    s = lax.dot_general(q, k, (((1,), (1,)), ((), ())),
                        preferred_element_type=jnp.float32)
    cols = lax.broadcasted_iota(jnp.int32, (R, C), 1)
    rows = r0 + lax.broadcasted_iota(jnp.int32, (R, 1), 0)
    s = jnp.where((cols == 0) | (cols == rows), NEG_MASK, s)

    vals, idxs = [], []
    for _ in range(K):
        m = jnp.max(s, axis=1, keepdims=True)
        hit = s == m
        sel = jnp.min(jnp.where(hit, cols, BIGI), axis=1, keepdims=True)
        vals.append(m)
        idxs.append(sel)
        s = jnp.where(hit & (cols == sel), NEG_INIT, s)

    tv = jnp.concatenate(vals, axis=1) / jnp.float32(0.1)
    mx = jnp.max(tv, axis=1, keepdims=True)
    e = jnp.exp(tv - mx)
    weights_ref[0] = e / jnp.sum(e, axis=1, keepdims=True)
    routes_ref[0] = jnp.concatenate(idxs, axis=1) - 1


def _topk_route(q, k, b):
    grid = (PP // R,)
    return pl.pallas_call(
        _topk_body,
        grid=grid,
        in_specs=[
            pl.BlockSpec((1, R, D), lambda rt: (b, rt, 0)),
            pl.BlockSpec((1, C, D), lambda rt: (b, 0, 0)),
        ],
        out_specs=[
            pl.BlockSpec((1, R, K), lambda rt: (0, rt, 0)),
            pl.BlockSpec((1, R, K), lambda rt: (0, rt, 0)),
        ],
        out_shape=[
            jax.ShapeDtypeStruct((1, PP, K), jnp.int32),
            jax.ShapeDtypeStruct((1, PP, K), jnp.float32),
        ],
    )(q, k)


SC_ROWS = 4096
SC_NW = 32
SC_RPW = SC_ROWS // SC_NW
SC_CH = 8
SC_NCH = SC_RPW // SC_CH


def _lane_bcast(vec, lane):
    idx = jnp.full((16, 1), lane, jnp.int32)
    dn = lax.GatherDimensionNumbers(
        offset_dims=(), collapsed_slice_dims=(0,), start_index_map=(0,))
    return lax.gather(vec, idx, dn, (1,),
                      mode=lax.GatherScatterMode.PROMISE_IN_BOUNDS)


def _sc_body(v_hbm, gidx_hbm, w_hbm, out_hbm,
             idx_all, w_all, rows_a, rows_b, acc_v, sem_a, sem_b):
    nc = 2
    wid = lax.axis_index("s") * nc + lax.axis_index("c")
    base_row = wid * SC_RPW
    ibase = base_row * K
    pltpu.sync_copy(gidx_hbm.at[pl.ds(ibase, SC_RPW * K)], idx_all)
    pltpu.sync_copy(w_hbm.at[pl.ds(ibase, SC_RPW * K)], w_all)

    def start_gather(ci, rows_ref, sem):
        pltpu.async_copy(
            v_hbm.at[idx_all.at[pl.ds(ci * (SC_CH * K), SC_CH * K)]],
            rows_ref, sem)

    def drain(rows_ref, sem):
        pltpu.make_async_copy(v_hbm.at[pl.ds(0, SC_CH * K)], rows_ref,
                              sem).wait()

    def compute(ci, rows_ref):
        def row_body(r, _):
            wvec = w_all[pl.ds(ci * (SC_CH * K) + (r // 2) * 16, 16)]
            half = (r % 2) * K
            wk = [_lane_bcast(wvec, half + kk) for kk in range(K)]

            def dc_body(dc, _):
                a = rows_ref[r * K, pl.ds(dc * 16, 16)] * wk[0]
                for kk in range(1, K):
                    a = a + rows_ref[r * K + kk, pl.ds(dc * 16, 16)] * wk[kk]
                acc_v[r, pl.ds(dc * 16, 16)] = a
                return 0

            lax.fori_loop(0, D // 16, dc_body, 0, unroll=4)
            return 0

        lax.fori_loop(0, SC_CH, row_body, 0)
        rbase = base_row + ci * SC_CH
        pltpu.sync_copy(acc_v, out_hbm.at[pl.ds(rbase, SC_CH)])

    start_gather(0, rows_a, sem_a)

    def pair_body(i, _):
        ca = 2 * i
        cb = 2 * i + 1
        start_gather(cb, rows_b, sem_b)
        drain(rows_a, sem_a)
        compute(ca, rows_a)
        nxt = lax.rem(ca + 2, SC_NCH)
        start_gather(nxt, rows_a, sem_a)
        drain(rows_b, sem_b)
        compute(cb, rows_b)
        return 0

    lax.fori_loop(0, SC_NCH // 2, pair_body, 0)
    drain(rows_a, sem_a)


def _sc_combine(v_flat, gidx, w_flat):
    mesh = plsc.VectorSubcoreMesh(core_axis_name="c", subcore_axis_name="s")
    fn = functools.partial(
        pl.kernel,
        mesh=mesh,
        out_type=jax.ShapeDtypeStruct((SC_ROWS, D), jnp.float32),
        scratch_types=[
            pltpu.VMEM((SC_RPW * K,), jnp.int32),
            pltpu.VMEM((SC_RPW * K,), jnp.float32),
            pltpu.VMEM((SC_CH * K, D), jnp.float32),
            pltpu.VMEM((SC_CH * K, D), jnp.float32),
            pltpu.VMEM((SC_CH, D), jnp.float32),
            pltpu.SemaphoreType.DMA,
            pltpu.SemaphoreType.DMA,
        ],
    )(_sc_body)
    return fn(v_flat, gidx, w_flat)


def kernel(x, Wq, bq, Wk, bk, Wv, bv):
    q, k, v = _project(x, Wq, bq, Wk, bk, Wv, bv)
    v_flat = v.reshape(B * PP, D)

    routes_b, weights_b, feats_b = [], [], []
    for b in range(B):
        r_full, w_full = _topk_route(q, k, b)
        routes1 = r_full[0, 1:, :]
        weights1 = w_full[0, 1:, :]
        gidx = jnp.pad((routes1 + (b * PP + 1)).reshape(-1), (0, K))
        w_flat = jnp.pad(weights1.reshape(-1), (0, K))
        feat = _sc_combine(v_flat, gidx, w_flat)
        routes_b.append(routes1)
        weights_b.append(weights1)
        feats_b.append(feat[: PP - 1])

    routes = jnp.stack(routes_b)
    weights = jnp.stack(weights_b)
    features = jnp.stack(feats_b)
    return (routes, weights, features)
